# baseline (speedup 1.0000x reference)
import functools

import jax
import jax.numpy as jnp
from jax import lax
from jax.experimental import pallas as pl
from jax.experimental.pallas import tpu as pltpu
from jax.experimental.pallas import tpu_sc as plsc

B = 32
L_SEQ = 4096
L_TREE = L_SEQ - 1
DEPTH = 12
D_FEAT = 128
LANES = 16
NGRP = D_FEAT // LANES
N_LEAVES = 2 ** (DEPTH - 1)
CHUNK = 256
NCHUNK = N_LEAVES // CHUNK


def _tanh(t):
    return 1.0 - 2.0 / (1.0 + jnp.exp(t + t))


def _pair_reduce(src_ref, src_base, dst_ref, dst_base, n_out):
    def body(j, carry):
        for k in range(NGRP):
            sl = pl.ds(LANES * k, LANES)
            a = src_ref[src_base + 2 * j, sl]
            b = src_ref[src_base + 2 * j + 1, sl]
            dst_ref[dst_base + j, sl] = _tanh(a + b)
        return carry

    lax.fori_loop(0, n_out, body, 0)


_INTL_OFF = {10: 0, 9: 128, 8: 192, 7: 224, 6: 240, 5: 248, 4: 252}
_INTL_ROWS = 254


def _sc_body(seq_hbm, out_hbm, inbuf, intl, roots, top, tail):
    wid = lax.axis_index("s") * 2 + lax.axis_index("c")
    b = wid

    def chunk_body(c, carry):
        leaf_start = (N_LEAVES - 1) + c * CHUNK
        pltpu.sync_copy(seq_hbm.at[b, pl.ds(leaf_start, CHUNK)], inbuf)
        pltpu.sync_copy(inbuf, out_hbm.at[b, pl.ds(leaf_start, CHUNK)])
        _pair_reduce(inbuf, 0, intl, _INTL_OFF[10], 128)
        _pair_reduce(intl, _INTL_OFF[10], intl, _INTL_OFF[9], 64)
        _pair_reduce(intl, _INTL_OFF[9], intl, _INTL_OFF[8], 32)
        _pair_reduce(intl, _INTL_OFF[8], intl, _INTL_OFF[7], 16)
        _pair_reduce(intl, _INTL_OFF[7], intl, _INTL_OFF[6], 8)
        _pair_reduce(intl, _INTL_OFF[6], intl, _INTL_OFF[5], 4)
        _pair_reduce(intl, _INTL_OFF[5], intl, _INTL_OFF[4], 2)
        _pair_reduce(intl, _INTL_OFF[4], roots, c, 1)
        for lvl in range(10, 3, -1):
            cnt = 2 ** (lvl - 3)
            pltpu.sync_copy(
                intl.at[pl.ds(_INTL_OFF[lvl], cnt)],
                out_hbm.at[b, pl.ds((2 ** lvl - 1) + c * cnt, cnt)],
            )
        return carry

    lax.fori_loop(0, NCHUNK, chunk_body, 0)

    _pair_reduce(roots, 0, top, 3, 4)
    _pair_reduce(top, 3, top, 1, 2)
    _pair_reduce(top, 1, top, 0, 1)
    pltpu.sync_copy(top, out_hbm.at[b, pl.ds(0, 7)])
    pltpu.sync_copy(roots, out_hbm.at[b, pl.ds(7, 8)])

    pltpu.sync_copy(seq_hbm.at[b, pl.ds(L_SEQ - 1, 1)], tail)
    pltpu.sync_copy(tail, out_hbm.at[b, pl.ds(L_SEQ - 1, 1)])


def kernel(seqs, parent_idx, node_level):
    del parent_idx, node_level
    mesh = plsc.VectorSubcoreMesh(core_axis_name="c", subcore_axis_name="s")
    run = functools.partial(
        pl.kernel,
        out_type=jax.ShapeDtypeStruct((B, L_SEQ, D_FEAT), jnp.float32),
        mesh=mesh,
        scratch_types=[
            pltpu.VMEM((CHUNK, D_FEAT), jnp.float32),
            pltpu.VMEM((_INTL_ROWS, D_FEAT), jnp.float32),
            pltpu.VMEM((NCHUNK, D_FEAT), jnp.float32),
            pltpu.VMEM((7, D_FEAT), jnp.float32),
            pltpu.VMEM((1, D_FEAT), jnp.float32),
        ],
        compiler_params=pltpu.CompilerParams(use_tc_tiling_on_sc=False),
    )(_sc_body)
    return run(seqs)


# speedup vs baseline: 7.5942x; 7.5942x over previous
import functools

import jax
import jax.numpy as jnp
from jax import lax
from jax.experimental import pallas as pl
from jax.experimental.pallas import tpu as pltpu
from jax.experimental.pallas import tpu_sc as plsc

B = 32
L_SEQ = 4096
L_TREE = L_SEQ - 1
DEPTH = 12
D_FEAT = 128
LANES = 16
NGRP = D_FEAT // LANES
N_LEAVES = 2 ** (DEPTH - 1)
CHUNK = 128
NCHUNK = N_LEAVES // CHUNK

_INTL_OFF = {10: 0, 9: 64, 8: 96, 7: 112, 6: 120, 5: 124}
_INTL_ROWS = 126
_TOP_ROWS = 31


def _tanh(t):
    return 1.0 - 2.0 / (1.0 + jnp.exp(t + t))


def _pair_reduce(src_ref, src_pre, src_base, dst_ref, dst_pre, dst_base,
                 n_out, unroll=1):
    @functools.partial(plsc.parallel_loop, 0, n_out, unroll=unroll)
    def _(j):
        for k in range(NGRP):
            sl = pl.ds(LANES * k, LANES)
            a = src_ref[(*src_pre, src_base + 2 * j, sl)]
            b = src_ref[(*src_pre, src_base + 2 * j + 1, sl)]
            dst_ref[(*dst_pre, dst_base + j, sl)] = _tanh(a + b)


def _chunk_levels(c):
    out = []
    for lvl in range(10, 4, -1):
        cnt = 2 ** (lvl - 4)
        out.append((_INTL_OFF[lvl], cnt, (2 ** lvl - 1) + c * cnt))
    return out


def _sc_body(seq_hbm, out_hbm, inbuf, intl, top, tail, rsem, psem, lsem, tsem):
    wid = lax.axis_index("s") * 2 + lax.axis_index("c")
    b = wid

    def leaf_slice(c):
        return seq_hbm.at[b, pl.ds((N_LEAVES - 1) + c * CHUNK, CHUNK)]

    def pass_slice(c):
        return out_hbm.at[b, pl.ds((N_LEAVES - 1) + c * CHUNK, CHUNK)]

    pltpu.async_copy(leaf_slice(0), inbuf.at[0], rsem)
    pltpu.async_copy(seq_hbm.at[b, pl.ds(L_SEQ - 1, 1)], tail, tsem)

    def chunk_step(c, carry):
        s = lax.rem(c, 2)
        pltpu.make_async_copy(leaf_slice(c), inbuf.at[s], rsem).wait()

        @pl.when(c < NCHUNK - 1)
        def _prefetch():
            s1 = lax.rem(c + 1, 2)

            @pl.when(c >= 1)
            def _slot_free():
                pltpu.make_async_copy(inbuf.at[s1], pass_slice(c - 1),
                                      psem).wait()

            pltpu.async_copy(leaf_slice(c + 1), inbuf.at[s1], rsem)

        @pl.when(c >= 2)
        def _intl_free():
            for off, cnt, base0 in _chunk_levels(0):
                hbm_base = base0 + (c - 2) * cnt
                pltpu.make_async_copy(
                    intl.at[s, pl.ds(off, cnt)],
                    out_hbm.at[b, pl.ds(hbm_base, cnt)], lsem).wait()

        _pair_reduce(inbuf, (s,), 0, intl, (s,), _INTL_OFF[10], 64, unroll=2)
        _pair_reduce(intl, (s,), _INTL_OFF[10], intl, (s,), _INTL_OFF[9], 32,
                     unroll=2)
        _pair_reduce(intl, (s,), _INTL_OFF[9], intl, (s,), _INTL_OFF[8], 16,
                     unroll=2)
        _pair_reduce(intl, (s,), _INTL_OFF[8], intl, (s,), _INTL_OFF[7], 8)
        _pair_reduce(intl, (s,), _INTL_OFF[7], intl, (s,), _INTL_OFF[6], 4)
        _pair_reduce(intl, (s,), _INTL_OFF[6], intl, (s,), _INTL_OFF[5], 2)
        _pair_reduce(intl, (s,), _INTL_OFF[5], top, (), 15 + c, 1)

        pltpu.async_copy(inbuf.at[s], pass_slice(c), psem)
        for off, cnt, hbm_base in _chunk_levels(c):
            pltpu.async_copy(intl.at[s, pl.ds(off, cnt)],
                             out_hbm.at[b, pl.ds(hbm_base, cnt)], lsem)
        return carry

    lax.fori_loop(0, NCHUNK, chunk_step, 0)

    for c in (NCHUNK - 2, NCHUNK - 1):
        s = c % 2
        pltpu.make_async_copy(inbuf.at[s], pass_slice(c), psem).wait()
        for off, cnt, hbm_base in _chunk_levels(c):
            pltpu.make_async_copy(intl.at[s, pl.ds(off, cnt)],
                                  out_hbm.at[b, pl.ds(hbm_base, cnt)],
                                  lsem).wait()

    _pair_reduce(top, (), 15, top, (), 7, 8)
    _pair_reduce(top, (), 7, top, (), 3, 4)
    _pair_reduce(top, (), 3, top, (), 1, 2)
    _pair_reduce(top, (), 1, top, (), 0, 1)
    pltpu.sync_copy(top, out_hbm.at[b, pl.ds(0, _TOP_ROWS)])

    pltpu.make_async_copy(seq_hbm.at[b, pl.ds(L_SEQ - 1, 1)], tail,
                          tsem).wait()
    pltpu.sync_copy(tail, out_hbm.at[b, pl.ds(L_SEQ - 1, 1)])


def kernel(seqs, parent_idx, node_level):
    del parent_idx, node_level
    mesh = plsc.VectorSubcoreMesh(core_axis_name="c", subcore_axis_name="s")
    run = functools.partial(
        pl.kernel,
        out_type=jax.ShapeDtypeStruct((B, L_SEQ, D_FEAT), jnp.float32),
        mesh=mesh,
        scratch_types=[
            pltpu.VMEM((2, CHUNK, D_FEAT), jnp.float32),
            pltpu.VMEM((2, _INTL_ROWS, D_FEAT), jnp.float32),
            pltpu.VMEM((_TOP_ROWS, D_FEAT), jnp.float32),
            pltpu.VMEM((1, D_FEAT), jnp.float32),
            pltpu.SemaphoreType.DMA,
            pltpu.SemaphoreType.DMA,
            pltpu.SemaphoreType.DMA,
            pltpu.SemaphoreType.DMA,
        ],
        compiler_params=pltpu.CompilerParams(use_tc_tiling_on_sc=False),
    )(_sc_body)
    return run(seqs)


# speedup vs baseline: 7.6740x; 1.0105x over previous
import functools

import jax
import jax.numpy as jnp
from jax import lax
from jax.experimental import pallas as pl
from jax.experimental.pallas import tpu as pltpu
from jax.experimental.pallas import tpu_sc as plsc

B = 32
L_SEQ = 4096
L_TREE = L_SEQ - 1
DEPTH = 12
D_FEAT = 128
LANES = 16
NGRP = D_FEAT // LANES
N_LEAVES = 2 ** (DEPTH - 1)
CHUNK = 128
NCHUNK = N_LEAVES // CHUNK

_INTL_OFF = {10: 0, 9: 64, 8: 96}
_INTL_ROWS = 112
_TOP_ROWS = 255


def _tanh(t):
    return 1.0 - 2.0 / (1.0 + jnp.exp(t + t))


def _pair_reduce(src_ref, src_pre, src_base, dst_ref, dst_pre, dst_base,
                 n_out, unroll=1):
    @functools.partial(plsc.parallel_loop, 0, n_out, unroll=unroll)
    def _(j):
        for k in range(NGRP):
            sl = pl.ds(LANES * k, LANES)
            a = src_ref[(*src_pre, src_base + 2 * j, sl)]
            b = src_ref[(*src_pre, src_base + 2 * j + 1, sl)]
            dst_ref[(*dst_pre, dst_base + j, sl)] = _tanh(a + b)


def _chunk_levels(c):
    out = []
    for lvl in (10, 9, 8):
        cnt = 2 ** (lvl - 4)
        out.append((_INTL_OFF[lvl], cnt, (2 ** lvl - 1) + c * cnt))
    return out


def _sc_body(seq_hbm, out_hbm, inbuf, intl, top, tail, rsem, psem, lsem, tsem):
    wid = lax.axis_index("s") * 2 + lax.axis_index("c")
    b = wid

    def leaf_slice(c):
        return seq_hbm.at[b, pl.ds((N_LEAVES - 1) + c * CHUNK, CHUNK)]

    def pass_slice(c):
        return out_hbm.at[b, pl.ds((N_LEAVES - 1) + c * CHUNK, CHUNK)]

    pltpu.async_copy(leaf_slice(0), inbuf.at[0], rsem)
    pltpu.async_copy(seq_hbm.at[b, pl.ds(L_SEQ - 1, 1)], tail, tsem)

    def chunk_step(c, carry):
        s = lax.rem(c, 2)
        pltpu.make_async_copy(leaf_slice(c), inbuf.at[s], rsem).wait()

        @pl.when(c < NCHUNK - 1)
        def _prefetch():
            s1 = lax.rem(c + 1, 2)

            @pl.when(c >= 1)
            def _slot_free():
                pltpu.make_async_copy(inbuf.at[s1], pass_slice(c - 1),
                                      psem).wait()

            pltpu.async_copy(leaf_slice(c + 1), inbuf.at[s1], rsem)

        @pl.when(c >= 2)
        def _intl_free():
            for off, cnt, base0 in _chunk_levels(0):
                hbm_base = base0 + (c - 2) * cnt
                pltpu.make_async_copy(
                    intl.at[s, pl.ds(off, cnt)],
                    out_hbm.at[b, pl.ds(hbm_base, cnt)], lsem).wait()

        _pair_reduce(inbuf, (s,), 0, intl, (s,), _INTL_OFF[10], 64, unroll=2)
        _pair_reduce(intl, (s,), _INTL_OFF[10], intl, (s,), _INTL_OFF[9], 32,
                     unroll=2)
        _pair_reduce(intl, (s,), _INTL_OFF[9], intl, (s,), _INTL_OFF[8], 16,
                     unroll=2)
        _pair_reduce(intl, (s,), _INTL_OFF[8], top, (), 127 + 8 * c, 8)
        _pair_reduce(top, (), 127 + 8 * c, top, (), 63 + 4 * c, 4)
        _pair_reduce(top, (), 63 + 4 * c, top, (), 31 + 2 * c, 2)
        _pair_reduce(top, (), 31 + 2 * c, top, (), 15 + c, 1)

        pltpu.async_copy(inbuf.at[s], pass_slice(c), psem)
        for off, cnt, hbm_base in _chunk_levels(c):
            pltpu.async_copy(intl.at[s, pl.ds(off, cnt)],
                             out_hbm.at[b, pl.ds(hbm_base, cnt)], lsem)
        return carry

    lax.fori_loop(0, NCHUNK, chunk_step, 0)

    _pair_reduce(top, (), 15, top, (), 7, 8)
    _pair_reduce(top, (), 7, top, (), 3, 4)
    _pair_reduce(top, (), 3, top, (), 1, 2)
    _pair_reduce(top, (), 1, top, (), 0, 1)
    pltpu.async_copy(top, out_hbm.at[b, pl.ds(0, _TOP_ROWS)], psem)

    pltpu.make_async_copy(seq_hbm.at[b, pl.ds(L_SEQ - 1, 1)], tail,
                          tsem).wait()
    pltpu.async_copy(tail, out_hbm.at[b, pl.ds(L_SEQ - 1, 1)], tsem)

    for c in (NCHUNK - 2, NCHUNK - 1):
        s = c % 2
        pltpu.make_async_copy(inbuf.at[s], pass_slice(c), psem).wait()
        for off, cnt, hbm_base in _chunk_levels(c):
            pltpu.make_async_copy(intl.at[s, pl.ds(off, cnt)],
                                  out_hbm.at[b, pl.ds(hbm_base, cnt)],
                                  lsem).wait()
    pltpu.make_async_copy(top, out_hbm.at[b, pl.ds(0, _TOP_ROWS)],
                          psem).wait()
    pltpu.make_async_copy(tail, out_hbm.at[b, pl.ds(L_SEQ - 1, 1)],
                          tsem).wait()


def kernel(seqs, parent_idx, node_level):
    del parent_idx, node_level
    mesh = plsc.VectorSubcoreMesh(core_axis_name="c", subcore_axis_name="s")
    run = functools.partial(
        pl.kernel,
        out_type=jax.ShapeDtypeStruct((B, L_SEQ, D_FEAT), jnp.float32),
        mesh=mesh,
        scratch_types=[
            pltpu.VMEM((2, CHUNK, D_FEAT), jnp.float32),
            pltpu.VMEM((2, _INTL_ROWS, D_FEAT), jnp.float32),
            pltpu.VMEM((_TOP_ROWS, D_FEAT), jnp.float32),
            pltpu.VMEM((1, D_FEAT), jnp.float32),
            pltpu.SemaphoreType.DMA,
            pltpu.SemaphoreType.DMA,
            pltpu.SemaphoreType.DMA,
            pltpu.SemaphoreType.DMA,
        ],
        compiler_params=pltpu.CompilerParams(use_tc_tiling_on_sc=False),
    )(_sc_body)
    return run(seqs)
